# baseline (speedup 1.0000x reference)
import functools

import jax
import jax.numpy as jnp
from jax import lax
from jax.experimental import pallas as pl
from jax.experimental.pallas import tpu as pltpu
from jax.experimental.pallas import tpu_sc as plsc

_NW = 32
_CHUNK = 4096


def _emb_call(x1, w1, b, s, d, v):
    n = x1.shape[0]
    nchunks = n // _CHUNK
    d_per = d // _NW
    nb = b // 128
    runs = _CHUNK // 128

    mesh = plsc.VectorSubcoreMesh(core_axis_name="c", subcore_axis_name="s")

    @functools.partial(
        pl.kernel,
        mesh=mesh,
        out_type=jax.ShapeDtypeStruct((s, d // 8, nb, 8 * 128), jnp.float32),
        compiler_params=pltpu.CompilerParams(
            use_tc_tiling_on_sc=False, needs_layout_passes=False
        ),
        scratch_types=[
            pltpu.VMEM((v,), jnp.float32),
            pltpu.VMEM((2, _CHUNK), jnp.int32),
            pltpu.VMEM((2, runs, 128), jnp.float32),
            pltpu.SemaphoreType.DMA((2,)),
            pltpu.SemaphoreType.DMA((2,)),
        ],
    )
    def emb(x_hbm, w_hbm, out_hbm, wrow_v, idx_v, out_v, isem, osem):
        wid = lax.axis_index("s") * 2 + lax.axis_index("c")

        def start_idx(c, pp):
            pltpu.async_copy(
                x_hbm.at[pl.ds(c * _CHUNK, _CHUNK)], idx_v.at[pp], isem.at[pp]
            )

        def wait_idx(pp):
            pltpu.make_async_copy(
                x_hbm.at[pl.ds(0, _CHUNK)], idx_v.at[pp], isem.at[pp]
            ).wait()

        cps = b // _CHUNK

        def out_slice(c, dd):
            return out_hbm.at[
                c // cps,
                dd // 8,
                pl.ds((c % cps) * runs, runs),
                pl.ds(pl.multiple_of((dd % 8) * 128, 128), 128),
            ]

        def start_out(c, dd, pp):
            pltpu.async_copy(out_v.at[pp], out_slice(c, dd), osem.at[pp])

        def wait_out(c, dd, pp):
            pltpu.make_async_copy(out_v.at[pp], out_slice(c, dd), osem.at[pp]).wait()

        def gather_chunk(pp):
            @plsc.parallel_loop(0, _CHUNK, 16, unroll=32)
            def _(i):
                iv = idx_v[pp, pl.ds(i, 16)]
                out_v[pp, i // 128, pl.ds(i % 128, 16)] = plsc.load_gather(
                    wrow_v, [iv]
                )

        for dloc in range(d_per):
            dd = wid * d_per + dloc
            pltpu.sync_copy(w_hbm.at[pl.ds(dd * v, v)], wrow_v)
            start_idx(0, 0)
            start_idx(1, 1)

            def body(cc, carry):
                for pp in range(2):
                    c = cc * 2 + pp

                    wait_idx(pp)

                    @pl.when(cc > 0)
                    def _():
                        wait_out(c - 2, dd, pp)

                    gather_chunk(pp)

                    @pl.when(cc < nchunks // 2 - 1)
                    def _():
                        start_idx(c + 2, pp)

                    start_out(c, dd, pp)
                return carry

            lax.fori_loop(0, nchunks // 2, body, 0)
            wait_out(nchunks - 2, dd, 0)
            wait_out(nchunks - 1, dd, 1)

    return emb(x1, w1)


def kernel(x, weight):
    b, s = x.shape
    v, d = weight.shape
    x1 = x.T.astype(jnp.int32).reshape(-1)
    w1 = weight.T.reshape(-1)
    out4 = _emb_call(x1, w1, b, s, d, v)
    out5 = out4.reshape(s, d // 8, b // 128, 8, 128)
    return out5.transpose(2, 4, 0, 1, 3).reshape(b, s, d)
